# Initial kernel scaffold; baseline (speedup 1.0000x reference)
import jax
import jax.numpy as jnp
from jax.experimental import pallas as pl


def kernel(x, edge_index, homophily_cond, W0, b0, g0, beta0, hW0, hb0, W1, b1, g1, beta1, hW1, hb1, muW, mub, lvW, lvb):
    raise NotImplementedError("write your pallas kernel here")


# speedup vs baseline: 9.4666x; 9.4666x over previous
import functools

import jax
import jax.numpy as jnp
from jax import lax
from jax.experimental import pallas as pl
from jax.experimental.pallas import tpu as pltpu
from jax.experimental.pallas import tpu_sc as plsc

N = 10000
FEAT = 128
HID = 128
LAT = 64
HOMD = 3
EPS = 1e-5

NC = 2
NS = 16
NW = NC * NS
CH = 128
NCHUNK = 79
EPT = CH * NCHUNK
EP = EPT * NW
E = 320000
NPAD = 10240
RPT = NPAD // NS

_f32 = jnp.float32
_i32 = jnp.int32


def _mesh():
    return plsc.VectorSubcoreMesh(core_axis_name="c", subcore_axis_name="s")


def _deg_body(dst_hbm, zvec_hbm, ones_hbm, out_hbm, idx_v, ones_v, acc_sh):
    c = lax.axis_index("c")
    s = lax.axis_index("s")
    wid = c * NS + s
    pltpu.sync_copy(zvec_hbm.at[pl.ds(s * RPT, RPT)], acc_sh.at[pl.ds(s * RPT, RPT)])
    pltpu.sync_copy(ones_hbm, ones_v)
    plsc.subcore_barrier()
    base = wid * EPT

    def body(i, carry):
        pltpu.sync_copy(dst_hbm.at[pl.ds(base + i * CH, CH)], idx_v)
        pltpu.sync_copy(ones_v, acc_sh.at[idx_v], add=True)
        return carry

    lax.fori_loop(0, NCHUNK, body, 0)
    plsc.subcore_barrier()
    pltpu.sync_copy(acc_sh.at[pl.ds(s * RPT, RPT)], out_hbm.at[c, pl.ds(s * RPT, RPT)])


_deg_kernel = functools.partial(
    pl.kernel,
    out_type=jax.ShapeDtypeStruct((NC, NPAD), _f32),
    mesh=_mesh(),
    scratch_types=[
        pltpu.VMEM((CH,), _i32),
        pltpu.VMEM((CH,), _f32),
        pltpu.VMEM_SHARED((NPAD,), _f32),
    ],
)(_deg_body)


def _edge_body(z_hbm, src_hbm, dst_hbm, zmat_hbm, out_hbm,
               sidx_v, didx_v, rows_v, acc_sh, sem):
    c = lax.axis_index("c")
    s = lax.axis_index("s")
    wid = c * NS + s
    pltpu.sync_copy(zmat_hbm.at[pl.ds(s * RPT, RPT), :],
                    acc_sh.at[pl.ds(s * RPT, RPT), :])
    plsc.subcore_barrier()
    base = wid * EPT

    def body(i, carry):
        pltpu.sync_copy(src_hbm.at[pl.ds(base + i * CH, CH)], sidx_v)
        pltpu.async_copy(z_hbm.at[sidx_v], rows_v, sem).wait()
        pltpu.sync_copy(dst_hbm.at[pl.ds(base + i * CH, CH)], didx_v)
        pltpu.sync_copy(rows_v, acc_sh.at[didx_v], add=True)
        return carry

    lax.fori_loop(0, NCHUNK, body, 0)
    plsc.subcore_barrier()
    pltpu.sync_copy(acc_sh.at[pl.ds(s * RPT, RPT), :],
                    out_hbm.at[c, pl.ds(s * RPT, RPT), :])


_edge_kernel = functools.partial(
    pl.kernel,
    out_type=jax.ShapeDtypeStruct((NC, NPAD, HID), _f32),
    mesh=_mesh(),
    scratch_types=[
        pltpu.VMEM((CH,), _i32),
        pltpu.VMEM((CH,), _i32),
        pltpu.VMEM((CH, HID), _f32),
        pltpu.VMEM_SHARED((NPAD, HID), _f32),
        pltpu.SemaphoreType.DMA,
    ],
)(_edge_body)


def _dinv(deg0_ref, deg1_ref):
    deg = deg0_ref[...] + deg1_ref[...] + 1.0
    return lax.rsqrt(deg)


def _tc1_body(x_ref, w0_ref, deg0_ref, deg1_ref, out_ref):
    dinv = _dinv(deg0_ref, deg1_ref)
    z = jnp.dot(x_ref[...], w0_ref[...], preferred_element_type=_f32)
    out_ref[pl.ds(0, N), :] = z * dinv[:N]
    out_ref[pl.ds(N, NPAD - N), :] = jnp.zeros((NPAD - N, HID), _f32)


_tc1 = pl.pallas_call(
    _tc1_body, out_shape=jax.ShapeDtypeStruct((NPAD, HID), _f32))


def _layer_mid(acc0_ref, acc1_ref, zs_ref, deg0_ref, deg1_ref,
               b_ref, g_ref, beta_ref, homp_ref, hwp_ref, hb_ref):
    dinv = _dinv(deg0_ref, deg1_ref)[:N]
    acc = acc0_ref[pl.ds(0, N), :] + acc1_ref[pl.ds(0, N), :] + zs_ref[pl.ds(0, N), :]
    t = acc * dinv + b_ref[...][None, :]
    mean = jnp.mean(t, axis=0, keepdims=True)
    var = jnp.mean((t - mean) ** 2, axis=0, keepdims=True)
    hn = (t - mean) / jnp.sqrt(var + EPS) * g_ref[...][None, :] + beta_ref[...][None, :]
    r = jnp.maximum(hn, 0.0)
    homrow = jnp.sum(homp_ref[...] * hwp_ref[...], axis=0) + hb_ref[...]
    return r + homrow[None, :], dinv


def _tc2_body(acc0_ref, acc1_ref, zs_ref, deg0_ref, deg1_ref,
              b_ref, g_ref, beta_ref, homp_ref, hwp_ref, hb_ref,
              w1_ref, out_ref):
    h, dinv = _layer_mid(acc0_ref, acc1_ref, zs_ref, deg0_ref, deg1_ref,
                         b_ref, g_ref, beta_ref, homp_ref, hwp_ref, hb_ref)
    z = jnp.dot(h, w1_ref[...], preferred_element_type=_f32)
    out_ref[pl.ds(0, N), :] = z * dinv
    out_ref[pl.ds(N, NPAD - N), :] = jnp.zeros((NPAD - N, HID), _f32)


_tc2 = pl.pallas_call(
    _tc2_body, out_shape=jax.ShapeDtypeStruct((NPAD, HID), _f32))


def _tc3_body(acc0_ref, acc1_ref, zs_ref, deg0_ref, deg1_ref,
              b_ref, g_ref, beta_ref, homp_ref, hwp_ref, hb_ref,
              muwh_ref, muwp_ref, mub_ref, lvwh_ref, lvwp_ref, lvb_ref,
              mu_ref, lv_ref):
    h, _ = _layer_mid(acc0_ref, acc1_ref, zs_ref, deg0_ref, deg1_ref,
                      b_ref, g_ref, beta_ref, homp_ref, hwp_ref, hb_ref)
    murow = jnp.sum(homp_ref[...] * muwp_ref[...], axis=0) + mub_ref[...]
    lvrow = jnp.sum(homp_ref[...] * lvwp_ref[...], axis=0) + lvb_ref[...]
    mu_ref[...] = jnp.dot(h, muwh_ref[...], preferred_element_type=_f32) + murow[None, :]
    lv_ref[...] = jnp.dot(h, lvwh_ref[...], preferred_element_type=_f32) + lvrow[None, :]


_tc3 = pl.pallas_call(
    _tc3_body,
    out_shape=(jax.ShapeDtypeStruct((N, LAT), _f32),
               jax.ShapeDtypeStruct((N, LAT), _f32)))


def kernel(x, edge_index, homophily_cond, W0, b0, g0, beta0, hW0, hb0,
           W1, b1, g1, beta1, hW1, hb1, muW, mub, lvW, lvb):
    pad = jnp.full((EP - E,), N, dtype=_i32)
    src = jnp.concatenate([edge_index[0].astype(_i32), pad])
    dst = jnp.concatenate([edge_index[1].astype(_i32), pad])
    zvec = jnp.zeros((NPAD,), _f32)
    ones = jnp.ones((CH,), _f32)
    zmat = jnp.zeros((NPAD, HID), _f32)
    homp = jnp.pad(homophily_cond, (0, 8 - HOMD)).reshape(8, 1)
    hw0p = jnp.pad(hW0, ((0, 8 - HOMD), (0, 0)))
    hw1p = jnp.pad(hW1, ((0, 8 - HOMD), (0, 0)))
    muwh, muwt = muW[:HID], jnp.pad(muW[HID:], ((0, 8 - HOMD), (0, 0)))
    lvwh, lvwt = lvW[:HID], jnp.pad(lvW[HID:], ((0, 8 - HOMD), (0, 0)))

    degp = _deg_kernel(dst, zvec, ones)
    deg0 = degp[0].reshape(NPAD, 1)
    deg1 = degp[1].reshape(NPAD, 1)

    z0s = _tc1(x, W0, deg0, deg1)
    acc1p = _edge_kernel(z0s, src, dst, zmat)
    z1s = _tc2(acc1p[0], acc1p[1], z0s, deg0, deg1,
               b0, g0, beta0, homp, hw0p, hb0, W1)
    acc2p = _edge_kernel(z1s, src, dst, zmat)
    mu, lv = _tc3(acc2p[0], acc2p[1], z1s, deg0, deg1,
                  b1, g1, beta1, homp, hw1p, hb1,
                  muwh, muwt, mub, lvwh, lvwt, lvb)
    return (mu, lv)
